# baseline (speedup 1.0000x reference)
import functools

import jax
import jax.numpy as jnp
from jax import lax
from jax.experimental import pallas as pl
from jax.experimental.pallas import tpu as pltpu
from jax.experimental.pallas import tpu_sc as plsc

B = 16
L = 2048
D = 128
NEXT_TOKEN_ID = 5
EPS = 1e-05

N = B * L
NC = 2
NS = 16
NW = NC * NS
PER_W = N // NW
CHUNK = 128
N_CHUNKS = PER_W // CHUNK


def _shift_right(x, s, fill):
    pad = jnp.full((x.shape[0], s), fill, dtype=x.dtype)
    return jnp.concatenate([pad, x[:, : x.shape[1] - s]], axis=1)


def _indices_kernel(state_ref, instr_ref, arg_ref):
    state = state_ref[...]
    eq = (state == NEXT_TOKEN_ID).astype(jnp.int32)
    csum = eq
    s = 1
    while s < L:
        csum = csum + _shift_right(csum, s, 0)
        s *= 2
    instr_ref[...] = csum - eq
    pos = lax.broadcasted_iota(jnp.int32, (B, L), 1)
    m = jnp.where(eq > 0, pos + 1, 0)
    s = 1
    while s < L:
        m = jnp.maximum(m, _shift_right(m, s, 0))
        s *= 2
    seg_start = _shift_right(m, 1, 0)
    arg_ref[...] = pos - seg_start


def _compute_indices(state):
    return pl.pallas_call(
        _indices_kernel,
        out_shape=(
            jax.ShapeDtypeStruct((B, L), jnp.int32),
            jax.ShapeDtypeStruct((B, L), jnp.int32),
        ),
    )(state)


NSLOT = 6


def _gather_sum_kernel(state_hbm, instr_hbm, arg_hbm,
                       tok_tab, ins_tab, arg_tab, out_hbm,
                       sidx, iidx, aidx, accs, gsems, osems):
    wid = lax.axis_index("s") * NC + lax.axis_index("c")
    base = wid * PER_W
    pltpu.sync_copy(state_hbm.at[pl.ds(base, PER_W)], sidx)
    pltpu.sync_copy(instr_hbm.at[pl.ds(base, PER_W)], iidx)
    pltpu.sync_copy(arg_hbm.at[pl.ds(base, PER_W)], aidx)

    def sl(c):
        return c % NSLOT

    pend = {}
    scat = {}
    for t in range(N_CHUNKS + 3):
        c0, c1, c2, c3 = t, t - 1, t - 2, t - 3
        if c0 < N_CHUNKS:
            if c0 >= NSLOT:
                scat.pop(c0 - NSLOT).wait()
            pend[c0] = pltpu.async_copy(
                tok_tab.at[sidx.at[pl.ds(c0 * CHUNK, CHUNK)]],
                accs[sl(c0)], gsems[sl(c0)])
        if 0 <= c1 < N_CHUNKS:
            pend.pop(c1).wait()
            pend[c1] = pltpu.async_copy(
                ins_tab.at[iidx.at[pl.ds(c1 * CHUNK, CHUNK)]],
                accs[sl(c1)], gsems[sl(c1)], add=True)
        if 0 <= c2 < N_CHUNKS:
            pend.pop(c2).wait()
            pend[c2] = pltpu.async_copy(
                arg_tab.at[aidx.at[pl.ds(c2 * CHUNK, CHUNK)]],
                accs[sl(c2)], gsems[sl(c2)], add=True)
        if 0 <= c3 < N_CHUNKS:
            pend.pop(c3).wait()
            scat[c3] = pltpu.async_copy(
                accs[sl(c3)], out_hbm.at[pl.ds(base + c3 * CHUNK, CHUNK)],
                osems[sl(c3)])
    for c in sorted(scat):
        scat[c].wait()


_gather_sum = functools.partial(
    pl.kernel,
    out_type=jax.ShapeDtypeStruct((N, D), jnp.float32),
    mesh=plsc.VectorSubcoreMesh(core_axis_name="c", subcore_axis_name="s"),
    scratch_types=[
        pltpu.VMEM((PER_W,), jnp.int32),
        pltpu.VMEM((PER_W,), jnp.int32),
        pltpu.VMEM((PER_W,), jnp.int32),
        [pltpu.VMEM((CHUNK, D), jnp.float32) for _ in range(NSLOT)],
        [pltpu.SemaphoreType.DMA for _ in range(NSLOT)],
        [pltpu.SemaphoreType.DMA for _ in range(NSLOT)],
    ],
)(_gather_sum_kernel)


LN_BLOCK = 1024


def _ln_kernel(x_ref, w_ref, b_ref, o_ref):
    x = x_ref[...]
    mean = jnp.mean(x, axis=-1, keepdims=True)
    d = x - mean
    var = jnp.mean(d * d, axis=-1, keepdims=True)
    rstd = lax.rsqrt(var + EPS)
    o_ref[...] = d * rstd * w_ref[...] + b_ref[...]


def _layernorm(x, w, b):
    return pl.pallas_call(
        _ln_kernel,
        grid=(N // LN_BLOCK,),
        in_specs=[
            pl.BlockSpec((LN_BLOCK, D), lambda i: (i, 0)),
            pl.BlockSpec((1, D), lambda i: (0, 0)),
            pl.BlockSpec((1, D), lambda i: (0, 0)),
        ],
        out_specs=pl.BlockSpec((LN_BLOCK, D), lambda i: (i, 0)),
        out_shape=jax.ShapeDtypeStruct((N, D), jnp.float32),
    )(x, w.reshape(1, D), b.reshape(1, D))


def kernel(state, token_table, instr_table, arg_table, ln_weight, ln_bias):
    instructions, arguments = _compute_indices(state)
    summed = _gather_sum(
        state.reshape(N), instructions.reshape(N), arguments.reshape(N),
        token_table, instr_table, arg_table)
    out = _layernorm(summed, ln_weight, ln_bias)
    return out.reshape(B, L, D)


# speedup vs baseline: 1.0033x; 1.0033x over previous
import functools

import jax
import jax.numpy as jnp
from jax import lax
from jax.experimental import pallas as pl
from jax.experimental.pallas import tpu as pltpu
from jax.experimental.pallas import tpu_sc as plsc

B = 16
L = 2048
D = 128
NEXT_TOKEN_ID = 5
EPS = 1e-05

N = B * L
NC = 2
NS = 16
NW = NC * NS
PER_W = N // NW
CHUNK = 128
N_CHUNKS = PER_W // CHUNK


def _shift_right(x, s, fill):
    pad = jnp.full((x.shape[0], s), fill, dtype=x.dtype)
    return jnp.concatenate([pad, x[:, : x.shape[1] - s]], axis=1)


def _indices_kernel(state_ref, instr_ref, arg_ref):
    state = state_ref[...]
    eq = (state == NEXT_TOKEN_ID).astype(jnp.int32)
    csum = eq
    s = 1
    while s < L:
        csum = csum + _shift_right(csum, s, 0)
        s *= 2
    instr_ref[...] = csum - eq
    pos = lax.broadcasted_iota(jnp.int32, (B, L), 1)
    m = jnp.where(eq > 0, pos + 1, 0)
    s = 1
    while s < L:
        m = jnp.maximum(m, _shift_right(m, s, 0))
        s *= 2
    seg_start = _shift_right(m, 1, 0)
    arg_ref[...] = pos - seg_start


def _compute_indices(state):
    return pl.pallas_call(
        _indices_kernel,
        out_shape=(
            jax.ShapeDtypeStruct((B, L), jnp.int32),
            jax.ShapeDtypeStruct((B, L), jnp.int32),
        ),
    )(state)


NSLOT = 2


def _gather_sum_kernel(state_hbm, instr_hbm, arg_hbm,
                       tok_tab, ins_tab, arg_tab, out_hbm,
                       sidx, iidx, aidx, tbufs, ibufs, abufs, gsems, osems):
    wid = lax.axis_index("s") * NC + lax.axis_index("c")
    base = pl.multiple_of(wid * PER_W, 256)
    pltpu.sync_copy(state_hbm.at[pl.ds(base, PER_W)], sidx)
    pltpu.sync_copy(instr_hbm.at[pl.ds(base, PER_W)], iidx)
    pltpu.sync_copy(arg_hbm.at[pl.ds(base, PER_W)], aidx)

    scat = {}

    def fire(c):
        s = c % NSLOT
        t = pltpu.async_copy(
            tok_tab.at[sidx.at[pl.ds(c * CHUNK, CHUNK)]], tbufs[s], gsems[s])
        i = pltpu.async_copy(
            ins_tab.at[iidx.at[pl.ds(c * CHUNK, CHUNK)]], ibufs[s], gsems[s])
        a = pltpu.async_copy(
            arg_tab.at[aidx.at[pl.ds(c * CHUNK, CHUNK)]], abufs[s], gsems[s])
        return (t, i, a)

    pend = {c: fire(c) for c in range(min(NSLOT, N_CHUNKS))}
    for c in range(N_CHUNKS):
        s = c % NSLOT
        for d in pend.pop(c):
            d.wait()

        def row(r, carry):
            for k in range(D // 16):
                tbufs[s][r, pl.ds(16 * k, 16)] = (
                    tbufs[s][r, pl.ds(16 * k, 16)]
                    + ibufs[s][r, pl.ds(16 * k, 16)]
                    + abufs[s][r, pl.ds(16 * k, 16)])
            return carry

        lax.fori_loop(0, CHUNK, row, jnp.int32(0))
        scat[c] = pltpu.async_copy(
            tbufs[s], out_hbm.at[pl.ds(base + c * CHUNK, CHUNK)], osems[s])
        if c + NSLOT < N_CHUNKS:
            scat.pop(c).wait()
            pend[c + NSLOT] = fire(c + NSLOT)
    for c in sorted(scat):
        scat[c].wait()


_gather_sum = functools.partial(
    pl.kernel,
    out_type=jax.ShapeDtypeStruct((N, D), jnp.float32),
    mesh=plsc.VectorSubcoreMesh(core_axis_name="c", subcore_axis_name="s"),
    scratch_types=[
        pltpu.VMEM((PER_W,), jnp.int32),
        pltpu.VMEM((PER_W,), jnp.int32),
        pltpu.VMEM((PER_W,), jnp.int32),
        [pltpu.VMEM((CHUNK, D), jnp.float32) for _ in range(NSLOT)],
        [pltpu.VMEM((CHUNK, D), jnp.float32) for _ in range(NSLOT)],
        [pltpu.VMEM((CHUNK, D), jnp.float32) for _ in range(NSLOT)],
        [pltpu.SemaphoreType.DMA for _ in range(NSLOT)],
        [pltpu.SemaphoreType.DMA for _ in range(NSLOT)],
    ],
)(_gather_sum_kernel)


LN_BLOCK = 1024


def _ln_kernel(x_ref, w_ref, b_ref, o_ref):
    x = x_ref[...]
    mean = jnp.mean(x, axis=-1, keepdims=True)
    d = x - mean
    var = jnp.mean(d * d, axis=-1, keepdims=True)
    rstd = lax.rsqrt(var + EPS)
    o_ref[...] = d * rstd * w_ref[...] + b_ref[...]


def _layernorm(x, w, b):
    return pl.pallas_call(
        _ln_kernel,
        grid=(N // LN_BLOCK,),
        in_specs=[
            pl.BlockSpec((LN_BLOCK, D), lambda i: (i, 0)),
            pl.BlockSpec((1, D), lambda i: (0, 0)),
            pl.BlockSpec((1, D), lambda i: (0, 0)),
        ],
        out_specs=pl.BlockSpec((LN_BLOCK, D), lambda i: (i, 0)),
        out_shape=jax.ShapeDtypeStruct((N, D), jnp.float32),
    )(x, w.reshape(1, D), b.reshape(1, D))


def kernel(state, token_table, instr_table, arg_table, ln_weight, ln_bias):
    instructions, arguments = _compute_indices(state)
    summed = _gather_sum(
        state.reshape(N), instructions.reshape(N), arguments.reshape(N),
        token_table, instr_table, arg_table)
    out = _layernorm(summed, ln_weight, ln_bias)
    return out.reshape(B, L, D)
